# baseline (speedup 1.0000x reference)
import functools

import jax
import jax.numpy as jnp
from jax import lax
from jax.experimental import pallas as pl
from jax.experimental.pallas import tpu as pltpu
from jax.experimental.pallas import tpu_sc as plsc

_B, _N, _C = 4, 1024, 768
_T = _B * _N
_NW = 32
_RPW = _T // _NW
_WPB = _N // _RPW


_RCH = 16
_NCH = _RPW // _RCH
_CG = _C // 16


def _sc_route_x1(im1f, im2f, m1, thr):
    mesh = plsc.VectorSubcoreMesh(core_axis_name="c", subcore_axis_name="s")

    _NSET = 3

    @functools.partial(
        pl.kernel,
        out_type=jax.ShapeDtypeStruct((_T, _C), jnp.float32),
        mesh=mesh,
        scratch_types=(
            [pltpu.VMEM((_RCH, _C), jnp.float32)] * _NSET
            + [pltpu.VMEM((_RCH, _C), jnp.float32)] * _NSET
            + [pltpu.VMEM((_RCH, _C), jnp.float32)] * _NSET
            + [pltpu.VMEM((_RPW,), jnp.float32),
               pltpu.VMEM((16,), jnp.float32)]
            + [pltpu.SemaphoreType.DMA] * (2 * _NSET)
        ),
        compiler_params=pltpu.CompilerParams(use_tc_tiling_on_sc=True),
    )
    def k(im1_hbm, im2_hbm, m1_hbm, thr_hbm, x1_hbm, *scr):
        av = scr[0:_NSET]
        bv = scr[_NSET:2 * _NSET]
        xv = scr[2 * _NSET:3 * _NSET]
        mv, tv = scr[3 * _NSET], scr[3 * _NSET + 1]
        sg = scr[3 * _NSET + 2:3 * _NSET + 2 + _NSET]
        ss = scr[3 * _NSET + 2 + _NSET:]

        wid = lax.axis_index("s") * 2 + lax.axis_index("c")
        t0 = wid * _RPW
        pltpu.sync_copy(m1_hbm.at[pl.ds(t0, _RPW)], mv)
        pltpu.sync_copy(thr_hbm, tv)
        tvec = tv[...]
        dnums = lax.GatherDimensionNumbers(
            offset_dims=(), collapsed_slice_dims=(0,), start_index_map=(0,))

        hg, hs = {}, {}

        def issue(c):
            s = c % _NSET
            sl = pl.ds(t0 + c * _RCH, _RCH)
            hg[c] = (pltpu.async_copy(im1_hbm.at[sl, :], av[s], sg[s]),
                     pltpu.async_copy(im2_hbm.at[sl, :], bv[s], sg[s]))

        def compute(c):
            s = c % _NSET
            kv = jnp.where(mv[pl.ds(c * _RCH, _RCH)] >= tvec,
                           jnp.full((16,), -1, jnp.int32),
                           jnp.full((16,), 0, jnp.int32))

            @plsc.parallel_loop(0, _RCH, 1, unroll=2)
            def _row(r):
                krow = lax.gather(
                    kv, jnp.full((16, 1), r, jnp.int32), dnums, (1,),
                    mode=lax.GatherScatterMode.PROMISE_IN_BOUNDS)
                nrow = ~krow
                for j in range(_CG):
                    a = lax.bitcast_convert_type(
                        av[s][r, pl.ds(j * 16, 16)], jnp.int32)
                    b = lax.bitcast_convert_type(
                        bv[s][r, pl.ds(j * 16, 16)], jnp.int32)
                    xv[s][r, pl.ds(j * 16, 16)] = lax.bitcast_convert_type(
                        (a & krow) | (b & nrow), jnp.float32)

        issue(0)
        issue(1)
        for c in range(_NCH):
            s = c % _NSET
            h1, h2 = hg.pop(c)
            h1.wait()
            h2.wait()
            compute(c)
            hs[c] = pltpu.async_copy(
                xv[s], x1_hbm.at[pl.ds(t0 + c * _RCH, _RCH), :], ss[s])
            if c + 2 < _NCH:
                if c - 1 >= 0:
                    hs.pop(c - 1).wait()
                issue(c + 2)
        for c in (_NCH - 2, _NCH - 1):
            hs.pop(c).wait()

    return k(im1f, im2f, m1, thr).reshape(_B, _N, _C)


_BLKN = 512


def _tc_body(thr_ref, m2_ref, a_ref, b_ref, x2_ref):
    t = thr_ref[0]
    k2 = m2_ref[...] >= t
    x2_ref[...] = jnp.where(k2, b_ref[...], a_ref[...])


def _tc_select_x2(im1, im2, m2col, thr):
    grid = (_B, _N // _BLKN)
    return pl.pallas_call(
        _tc_body,
        grid=grid,
        in_specs=[
            pl.BlockSpec(memory_space=pltpu.SMEM),
            pl.BlockSpec((1, _BLKN, 1), lambda i, j: (i, j, 0)),
            pl.BlockSpec((1, _BLKN, _C), lambda i, j: (i, j, 0)),
            pl.BlockSpec((1, _BLKN, _C), lambda i, j: (i, j, 0)),
        ],
        out_specs=pl.BlockSpec((1, _BLKN, _C), lambda i, j: (i, j, 0)),
        out_shape=jax.ShapeDtypeStruct((_B, _N, _C), jnp.float32),
        compiler_params=pltpu.CompilerParams(
            dimension_semantics=("arbitrary", "arbitrary")),
    )(thr, m2col, im1, im2)


def kernel(im1, im2, mask1, mask2, mask_threshold):
    m1f = mask1.reshape(_T)
    m2col = mask2.reshape(_B, _N, 1)
    thr16 = jnp.full((16,), mask_threshold, jnp.float32)
    thr1 = jnp.full((1,), mask_threshold, jnp.float32)
    x1 = _sc_route_x1(im1.reshape(_T, _C), im2.reshape(_T, _C), m1f, thr16)
    x2 = _tc_select_x2(im1, im2, m2col, thr1)
    return x1, x2


# speedup vs baseline: 1.0226x; 1.0226x over previous
import functools

import jax
import jax.numpy as jnp
from jax import lax
from jax.experimental import pallas as pl
from jax.experimental.pallas import tpu as pltpu
from jax.experimental.pallas import tpu_sc as plsc

_B, _N, _C = 4, 1024, 768
_T = _B * _N
_NW = 32
_RPW = _T // _NW
_WPB = _N // _RPW


_RCH = 16
_NCH = _RPW // _RCH
_CG = _C // 16


def _sc_route_x1(im1f, im2f, m1, thr):
    mesh = plsc.VectorSubcoreMesh(core_axis_name="c", subcore_axis_name="s")

    _NSET = 3

    @functools.partial(
        pl.kernel,
        out_type=jax.ShapeDtypeStruct((_T, _C), jnp.float32),
        mesh=mesh,
        scratch_types=(
            [pltpu.VMEM((_RCH, _C), jnp.float32)] * _NSET
            + [pltpu.VMEM((_RCH, _C), jnp.float32)] * _NSET
            + [pltpu.VMEM((_RCH, _C), jnp.float32)] * _NSET
            + [pltpu.VMEM((_RPW,), jnp.float32),
               pltpu.VMEM((16,), jnp.float32)]
            + [pltpu.SemaphoreType.DMA] * (2 * _NSET)
        ),
        compiler_params=pltpu.CompilerParams(use_tc_tiling_on_sc=True),
    )
    def k(im1_hbm, im2_hbm, m1_hbm, thr_hbm, x1_hbm, *scr):
        av = scr[0:_NSET]
        bv = scr[_NSET:2 * _NSET]
        xv = scr[2 * _NSET:3 * _NSET]
        mv, tv = scr[3 * _NSET], scr[3 * _NSET + 1]
        sg = scr[3 * _NSET + 2:3 * _NSET + 2 + _NSET]
        ss = scr[3 * _NSET + 2 + _NSET:]

        wid = lax.axis_index("s") * 2 + lax.axis_index("c")
        t0 = wid * _RPW
        pltpu.sync_copy(m1_hbm.at[pl.ds(t0, _RPW)], mv)
        pltpu.sync_copy(thr_hbm, tv)
        tvec = tv[...]
        dnums = lax.GatherDimensionNumbers(
            offset_dims=(), collapsed_slice_dims=(0,), start_index_map=(0,))

        hg, hs = {}, {}

        def issue(c):
            s = c % _NSET
            sl = pl.ds(t0 + c * _RCH, _RCH)
            hg[c] = (pltpu.async_copy(im1_hbm.at[sl, :], av[s], sg[s]),
                     pltpu.async_copy(im2_hbm.at[sl, :], bv[s], sg[s]))

        def compute(c):
            s = c % _NSET
            kv = jnp.where(mv[pl.ds(c * _RCH, _RCH)] >= tvec,
                           jnp.full((16,), -1, jnp.int32),
                           jnp.full((16,), 0, jnp.int32))

            def row(r, carry2):
                krow = lax.gather(
                    kv, jnp.full((16, 1), r, jnp.int32), dnums, (1,),
                    mode=lax.GatherScatterMode.PROMISE_IN_BOUNDS)
                nrow = ~krow
                for j in range(_CG):
                    a = lax.bitcast_convert_type(
                        av[s][r, pl.ds(j * 16, 16)], jnp.int32)
                    b = lax.bitcast_convert_type(
                        bv[s][r, pl.ds(j * 16, 16)], jnp.int32)
                    xv[s][r, pl.ds(j * 16, 16)] = lax.bitcast_convert_type(
                        (a & krow) | (b & nrow), jnp.float32)
                return carry2

            lax.fori_loop(0, _RCH, row, 0)

        issue(0)
        issue(1)
        for c in range(_NCH):
            s = c % _NSET
            h1, h2 = hg.pop(c)
            h1.wait()
            h2.wait()
            compute(c)
            hs[c] = pltpu.async_copy(
                xv[s], x1_hbm.at[pl.ds(t0 + c * _RCH, _RCH), :], ss[s])
            if c + 2 < _NCH:
                if c - 1 >= 0:
                    hs.pop(c - 1).wait()
                issue(c + 2)
        for c in (_NCH - 2, _NCH - 1):
            hs.pop(c).wait()

    return k(im1f, im2f, m1, thr).reshape(_B, _N, _C)


_BLKN = 512


def _tc_body(thr_ref, m2_ref, a_ref, b_ref, x2_ref):
    t = thr_ref[0]
    k2 = m2_ref[...] >= t
    x2_ref[...] = jnp.where(k2, b_ref[...], a_ref[...])


def _tc_select_x2(im1, im2, m2col, thr):
    grid = (_B, _N // _BLKN)
    return pl.pallas_call(
        _tc_body,
        grid=grid,
        in_specs=[
            pl.BlockSpec(memory_space=pltpu.SMEM),
            pl.BlockSpec((1, _BLKN, 1), lambda i, j: (i, j, 0)),
            pl.BlockSpec((1, _BLKN, _C), lambda i, j: (i, j, 0)),
            pl.BlockSpec((1, _BLKN, _C), lambda i, j: (i, j, 0)),
        ],
        out_specs=pl.BlockSpec((1, _BLKN, _C), lambda i, j: (i, j, 0)),
        out_shape=jax.ShapeDtypeStruct((_B, _N, _C), jnp.float32),
        compiler_params=pltpu.CompilerParams(
            dimension_semantics=("arbitrary", "arbitrary")),
    )(thr, m2col, im1, im2)


def kernel(im1, im2, mask1, mask2, mask_threshold):
    m1f = mask1.reshape(_T)
    m2col = mask2.reshape(_B, _N, 1)
    thr16 = jnp.full((16,), mask_threshold, jnp.float32)
    thr1 = jnp.full((1,), mask_threshold, jnp.float32)
    x1 = _sc_route_x1(im1.reshape(_T, _C), im2.reshape(_T, _C), m1f, thr16)
    x2 = _tc_select_x2(im1, im2, m2col, thr1)
    return x1, x2


# speedup vs baseline: 1.0325x; 1.0097x over previous
import functools

import jax
import jax.numpy as jnp
from jax import lax
from jax.experimental import pallas as pl
from jax.experimental.pallas import tpu as pltpu
from jax.experimental.pallas import tpu_sc as plsc

_B, _N, _C = 4, 1024, 768
_T = _B * _N
_NW = 32
_RPW = _T // _NW
_WPB = _N // _RPW


_RCH = 16
_NCH = _RPW // _RCH
_CG = _C // 16


def _sc_route_x1(im1f, im2f, m1, thr):
    mesh = plsc.VectorSubcoreMesh(core_axis_name="c", subcore_axis_name="s")

    _NSET = 3

    @functools.partial(
        pl.kernel,
        out_type=jax.ShapeDtypeStruct((_T, _C), jnp.float32),
        mesh=mesh,
        scratch_types=(
            [pltpu.VMEM((_RCH, _C), jnp.float32)] * _NSET
            + [pltpu.VMEM((_RCH, _C), jnp.float32)] * _NSET
            + [pltpu.VMEM((_RCH, _C), jnp.float32)] * _NSET
            + [pltpu.VMEM((_RPW,), jnp.float32),
               pltpu.VMEM((16,), jnp.float32)]
            + [pltpu.SemaphoreType.DMA] * (2 * _NSET)
        ),
        compiler_params=pltpu.CompilerParams(use_tc_tiling_on_sc=True),
    )
    def k(im1_hbm, im2_hbm, m1_hbm, thr_hbm, x1_hbm, *scr):
        av = scr[0:_NSET]
        bv = scr[_NSET:2 * _NSET]
        xv = scr[2 * _NSET:3 * _NSET]
        mv, tv = scr[3 * _NSET], scr[3 * _NSET + 1]
        sg = scr[3 * _NSET + 2:3 * _NSET + 2 + _NSET]
        ss = scr[3 * _NSET + 2 + _NSET:]

        wid = lax.axis_index("s") * 2 + lax.axis_index("c")
        t0 = wid * _RPW
        pltpu.sync_copy(
            m1_hbm.at[wid // _WPB, pl.ds((wid % _WPB) * _RPW, _RPW)], mv)
        pltpu.sync_copy(thr_hbm, tv)
        tvec = tv[...]
        dnums = lax.GatherDimensionNumbers(
            offset_dims=(), collapsed_slice_dims=(0,), start_index_map=(0,))

        hg, hs = {}, {}

        def issue(c):
            s = c % _NSET
            sl = pl.ds(t0 + c * _RCH, _RCH)
            hg[c] = (pltpu.async_copy(im1_hbm.at[sl, :], av[s], sg[s]),
                     pltpu.async_copy(im2_hbm.at[sl, :], bv[s], sg[s]))

        def compute(c):
            s = c % _NSET
            kv = jnp.where(mv[pl.ds(c * _RCH, _RCH)] >= tvec,
                           jnp.full((16,), -1, jnp.int32),
                           jnp.full((16,), 0, jnp.int32))

            def row(r, carry2):
                krow = lax.gather(
                    kv, jnp.full((16, 1), r, jnp.int32), dnums, (1,),
                    mode=lax.GatherScatterMode.PROMISE_IN_BOUNDS)
                nrow = ~krow
                for j in range(_CG):
                    a = lax.bitcast_convert_type(
                        av[s][r, pl.ds(j * 16, 16)], jnp.int32)
                    b = lax.bitcast_convert_type(
                        bv[s][r, pl.ds(j * 16, 16)], jnp.int32)
                    xv[s][r, pl.ds(j * 16, 16)] = lax.bitcast_convert_type(
                        (a & krow) | (b & nrow), jnp.float32)
                return carry2

            lax.fori_loop(0, _RCH, row, 0)

        issue(0)
        issue(1)
        for c in range(_NCH):
            s = c % _NSET
            h1, h2 = hg.pop(c)
            h1.wait()
            h2.wait()
            compute(c)
            hs[c] = pltpu.async_copy(
                xv[s], x1_hbm.at[pl.ds(t0 + c * _RCH, _RCH), :], ss[s])
            if c + 2 < _NCH:
                if c - 1 >= 0:
                    hs.pop(c - 1).wait()
                issue(c + 2)
        for c in (_NCH - 2, _NCH - 1):
            hs.pop(c).wait()

    return k(im1f, im2f, m1, thr).reshape(_B, _N, _C)


_BLKN = 512


def _tc_body(thr_ref, m2_ref, a_ref, b_ref, x2_ref):
    t = thr_ref[0]
    k2 = m2_ref[...] >= t
    x2_ref[...] = jnp.where(k2, b_ref[...], a_ref[...])


def _tc_select_x2(im1, im2, m2col, thr):
    grid = (_B, _N // _BLKN)
    return pl.pallas_call(
        _tc_body,
        grid=grid,
        in_specs=[
            pl.BlockSpec(memory_space=pltpu.SMEM),
            pl.BlockSpec((1, _BLKN, 1), lambda i, j: (i, j, 0)),
            pl.BlockSpec((1, _BLKN, _C), lambda i, j: (i, j, 0)),
            pl.BlockSpec((1, _BLKN, _C), lambda i, j: (i, j, 0)),
        ],
        out_specs=pl.BlockSpec((1, _BLKN, _C), lambda i, j: (i, j, 0)),
        out_shape=jax.ShapeDtypeStruct((_B, _N, _C), jnp.float32),
        compiler_params=pltpu.CompilerParams(
            dimension_semantics=("arbitrary", "arbitrary")),
    )(thr, m2col, im1, im2)


def kernel(im1, im2, mask1, mask2, mask_threshold):
    m2col = mask2.reshape(_B, _N, 1)
    thr16 = jnp.full((16,), mask_threshold, jnp.float32)
    thr1 = jnp.full((1,), mask_threshold, jnp.float32)
    x1 = _sc_route_x1(im1.reshape(_T, _C), im2.reshape(_T, _C), mask1, thr16)
    x2 = _tc_select_x2(im1, im2, m2col, thr1)
    return x1, x2


# speedup vs baseline: 1.1264x; 1.0909x over previous
import functools

import jax
import jax.numpy as jnp
from jax import lax
from jax.experimental import pallas as pl
from jax.experimental.pallas import tpu as pltpu
from jax.experimental.pallas import tpu_sc as plsc

_B, _N, _C = 4, 1024, 768
_T = _B * _N
_NW = 32
_RPW = _T // _NW
_WPB = _N // _RPW
_NG = _RPW // 16
_MAXBLK = _NG + 1


def _sc_route_x1(im1f, im2f, m1, thr):
    mesh = plsc.VectorSubcoreMesh(core_axis_name="c", subcore_axis_name="s")

    @functools.partial(
        pl.kernel,
        out_type=jax.ShapeDtypeStruct((_T, _C), jnp.float32),
        mesh=mesh,
        scratch_types=[
            pltpu.VMEM((_MAXBLK * 16, _C), jnp.float32),
            pltpu.VMEM((_RPW + 16,), jnp.int32),
            pltpu.VMEM((_RPW + 16,), jnp.int32),
            pltpu.VMEM((_MAXBLK, 16), jnp.int32),
            pltpu.VMEM((_MAXBLK, 16), jnp.int32),
            pltpu.VMEM((_RPW,), jnp.float32),
            pltpu.VMEM((16,), jnp.float32),
            pltpu.SemaphoreType.DMA,
            pltpu.SemaphoreType.DMA,
        ],
        compiler_params=pltpu.CompilerParams(use_tc_tiling_on_sc=True),
    )
    def k(im1_hbm, im2_hbm, m1_hbm, thr_hbm, x1_hbm,
          xbuf, ia1, ib1, ia2, ib2, mv, tv, semg, sems):
        wid = lax.axis_index("s") * 2 + lax.axis_index("c")
        t0 = wid * _RPW
        pltpu.sync_copy(
            m1_hbm.at[wid // _WPB, pl.ds((wid % _WPB) * _RPW, _RPW)], mv)
        pltpu.sync_copy(thr_hbm, tv)
        tvec = tv[...]

        lane = lax.iota(jnp.int32, 16)
        zero16 = jnp.full((16,), 0, jnp.int32)

        cnt_a = jnp.int32(0)
        cnt_b = jnp.int32(0)
        for g in range(_NG):
            keep = mv[pl.ds(g * 16, 16)] >= tvec
            ki = jnp.where(keep, jnp.full((16,), 1, jnp.int32), zero16)
            for j in range(16):
                kj = ki[j]
                tj = t0 + g * 16 + j

                @pl.when(kj > 0)
                def _():
                    ia1[pl.ds(cnt_a, 16)] = jnp.full((16,), tj, jnp.int32)

                @pl.when(kj == 0)
                def _():
                    ib1[pl.ds(cnt_b, 16)] = jnp.full((16,), tj, jnp.int32)

                cnt_a = cnt_a + kj
                cnt_b = cnt_b + (1 - kj)

        for blk in range(_MAXBLK):
            ia2[blk, :] = ia1[pl.ds(blk * 16, 16)]
            ib2[blk, :] = ib1[pl.ds(blk * 16, 16)]

        nblk_a = (cnt_a + 15) // 16
        nblk_b = (cnt_b + 15) // 16

        def gat_a(blk, carry):
            pltpu.async_copy(
                im1_hbm.at[ia2.at[blk]], xbuf.at[pl.ds(blk * 16, 16), :],
                semg)
            return carry

        lax.fori_loop(0, nblk_a, gat_a, 0)

        def gat_b(blk, carry):
            pltpu.async_copy(
                im2_hbm.at[ib2.at[blk]],
                xbuf.at[pl.ds((nblk_a + blk) * 16, 16), :], semg)
            return carry

        lax.fori_loop(0, nblk_b, gat_b, 0)

        def drain_g(i, carry):
            pltpu.make_async_copy(
                im1_hbm.at[pl.ds(0, 16), :], xbuf.at[pl.ds(0, 16), :],
                semg).wait()
            return carry

        lax.fori_loop(0, nblk_a + nblk_b, drain_g, 0)

        def sca_a(blk, carry):
            pltpu.async_copy(
                xbuf.at[pl.ds(blk * 16, 16), :], x1_hbm.at[ia2.at[blk]],
                sems)
            return carry

        lax.fori_loop(0, nblk_a, sca_a, 0)

        def sca_b(blk, carry):
            pltpu.async_copy(
                xbuf.at[pl.ds((nblk_a + blk) * 16, 16), :],
                x1_hbm.at[ib2.at[blk]], sems)
            return carry

        lax.fori_loop(0, nblk_b, sca_b, 0)

        def drain_s(i, carry):
            pltpu.make_async_copy(
                im1_hbm.at[pl.ds(0, 16), :], xbuf.at[pl.ds(0, 16), :],
                sems).wait()
            return carry

        lax.fori_loop(0, nblk_a + nblk_b, drain_s, 0)

    return k(im1f, im2f, m1, thr).reshape(_B, _N, _C)


_BLKN = 512


def _tc_body(thr_ref, m2_ref, a_ref, b_ref, x2_ref):
    t = thr_ref[0]
    k2 = m2_ref[...] >= t
    x2_ref[...] = jnp.where(k2, b_ref[...], a_ref[...])


def _tc_select_x2(im1, im2, m2col, thr):
    grid = (_B, _N // _BLKN)
    return pl.pallas_call(
        _tc_body,
        grid=grid,
        in_specs=[
            pl.BlockSpec(memory_space=pltpu.SMEM),
            pl.BlockSpec((1, _BLKN, 1), lambda i, j: (i, j, 0)),
            pl.BlockSpec((1, _BLKN, _C), lambda i, j: (i, j, 0)),
            pl.BlockSpec((1, _BLKN, _C), lambda i, j: (i, j, 0)),
        ],
        out_specs=pl.BlockSpec((1, _BLKN, _C), lambda i, j: (i, j, 0)),
        out_shape=jax.ShapeDtypeStruct((_B, _N, _C), jnp.float32),
        compiler_params=pltpu.CompilerParams(
            dimension_semantics=("arbitrary", "arbitrary")),
    )(thr, m2col, im1, im2)


def kernel(im1, im2, mask1, mask2, mask_threshold):
    m2col = mask2.reshape(_B, _N, 1)
    thr16 = jnp.full((16,), mask_threshold, jnp.float32)
    thr1 = jnp.full((1,), mask_threshold, jnp.float32)
    x1 = _sc_route_x1(im1.reshape(_T, _C), im2.reshape(_T, _C), mask1, thr16)
    x2 = _tc_select_x2(im1, im2, m2col, thr1)
    return x1, x2


# speedup vs baseline: 1.1335x; 1.0063x over previous
import functools

import jax
import jax.numpy as jnp
from jax import lax
from jax.experimental import pallas as pl
from jax.experimental.pallas import tpu as pltpu
from jax.experimental.pallas import tpu_sc as plsc

_B, _N, _C = 4, 1024, 768
_T = _B * _N
_NW = 32
_RPW = _T // _NW
_WPB = _N // _RPW
_NG = _RPW // 16
_MAXBLK = _NG + 1


def _sc_route_x1(im1f, im2f, m1, thr):
    mesh = plsc.VectorSubcoreMesh(core_axis_name="c", subcore_axis_name="s")

    @functools.partial(
        pl.kernel,
        out_type=jax.ShapeDtypeStruct((_T, _C), jnp.float32),
        mesh=mesh,
        scratch_types=[
            pltpu.VMEM((_MAXBLK * 16, _C), jnp.float32),
            pltpu.VMEM((_RPW + 16,), jnp.int32),
            pltpu.VMEM((_RPW + 16,), jnp.int32),
            pltpu.VMEM((_MAXBLK, 16), jnp.int32),
            pltpu.VMEM((_MAXBLK, 16), jnp.int32),
            pltpu.VMEM((_RPW,), jnp.float32),
            pltpu.VMEM((16,), jnp.float32),
            pltpu.SemaphoreType.DMA,
            pltpu.SemaphoreType.DMA,
            pltpu.SemaphoreType.DMA,
        ],
        compiler_params=pltpu.CompilerParams(use_tc_tiling_on_sc=True),
    )
    def k(im1_hbm, im2_hbm, m1_hbm, thr_hbm, x1_hbm,
          xbuf, ia1, ib1, ia2, ib2, mv, tv, semga, semgb, sems):
        wid = lax.axis_index("s") * 2 + lax.axis_index("c")
        t0 = wid * _RPW
        pltpu.sync_copy(
            m1_hbm.at[wid // _WPB, pl.ds((wid % _WPB) * _RPW, _RPW)], mv)
        pltpu.sync_copy(thr_hbm, tv)
        tvec = tv[...]

        lane = lax.iota(jnp.int32, 16)
        zero16 = jnp.full((16,), 0, jnp.int32)

        cnt_a = jnp.int32(0)
        cnt_b = jnp.int32(0)
        for g in range(_NG):
            keep = mv[pl.ds(g * 16, 16)] >= tvec
            ki = jnp.where(keep, jnp.full((16,), 1, jnp.int32), zero16)
            for j in range(16):
                kj = ki[j]
                tj = t0 + g * 16 + j

                @pl.when(kj > 0)
                def _():
                    ia1[pl.ds(cnt_a, 16)] = jnp.full((16,), tj, jnp.int32)

                @pl.when(kj == 0)
                def _():
                    ib1[pl.ds(cnt_b, 16)] = jnp.full((16,), tj, jnp.int32)

                cnt_a = cnt_a + kj
                cnt_b = cnt_b + (1 - kj)

        for blk in range(_MAXBLK):
            ia2[blk, :] = ia1[pl.ds(blk * 16, 16)]
            ib2[blk, :] = ib1[pl.ds(blk * 16, 16)]

        nblk_a = (cnt_a + 15) // 16
        nblk_b = (cnt_b + 15) // 16

        def gat_a(blk, carry):
            pltpu.async_copy(
                im1_hbm.at[ia2.at[blk]], xbuf.at[pl.ds(blk * 16, 16), :],
                semga)
            return carry

        lax.fori_loop(0, nblk_a, gat_a, 0)

        def gat_b(blk, carry):
            pltpu.async_copy(
                im2_hbm.at[ib2.at[blk]],
                xbuf.at[pl.ds((nblk_a + blk) * 16, 16), :], semgb)
            return carry

        lax.fori_loop(0, nblk_b, gat_b, 0)

        def drain_ga(i, carry):
            pltpu.make_async_copy(
                im1_hbm.at[pl.ds(0, 16), :], xbuf.at[pl.ds(0, 16), :],
                semga).wait()
            return carry

        lax.fori_loop(0, nblk_a, drain_ga, 0)

        def sca_a(blk, carry):
            pltpu.async_copy(
                xbuf.at[pl.ds(blk * 16, 16), :], x1_hbm.at[ia2.at[blk]],
                sems)
            return carry

        lax.fori_loop(0, nblk_a, sca_a, 0)

        def drain_gb(i, carry):
            pltpu.make_async_copy(
                im1_hbm.at[pl.ds(0, 16), :], xbuf.at[pl.ds(0, 16), :],
                semgb).wait()
            return carry

        lax.fori_loop(0, nblk_b, drain_gb, 0)

        def sca_b(blk, carry):
            pltpu.async_copy(
                xbuf.at[pl.ds((nblk_a + blk) * 16, 16), :],
                x1_hbm.at[ib2.at[blk]], sems)
            return carry

        lax.fori_loop(0, nblk_b, sca_b, 0)

        def drain_s(i, carry):
            pltpu.make_async_copy(
                im1_hbm.at[pl.ds(0, 16), :], xbuf.at[pl.ds(0, 16), :],
                sems).wait()
            return carry

        lax.fori_loop(0, nblk_a + nblk_b, drain_s, 0)

    return k(im1f, im2f, m1, thr).reshape(_B, _N, _C)


_BLKN = 512


def _tc_body(thr_ref, m2_ref, a_ref, b_ref, x2_ref):
    t = thr_ref[0]
    k2 = m2_ref[...] >= t
    x2_ref[...] = jnp.where(k2, b_ref[...], a_ref[...])


def _tc_select_x2(im1, im2, m2col, thr):
    grid = (_B, _N // _BLKN)
    return pl.pallas_call(
        _tc_body,
        grid=grid,
        in_specs=[
            pl.BlockSpec(memory_space=pltpu.SMEM),
            pl.BlockSpec((1, _BLKN, 1), lambda i, j: (i, j, 0)),
            pl.BlockSpec((1, _BLKN, _C), lambda i, j: (i, j, 0)),
            pl.BlockSpec((1, _BLKN, _C), lambda i, j: (i, j, 0)),
        ],
        out_specs=pl.BlockSpec((1, _BLKN, _C), lambda i, j: (i, j, 0)),
        out_shape=jax.ShapeDtypeStruct((_B, _N, _C), jnp.float32),
        compiler_params=pltpu.CompilerParams(
            dimension_semantics=("arbitrary", "arbitrary")),
    )(thr, m2col, im1, im2)


def kernel(im1, im2, mask1, mask2, mask_threshold):
    m2col = mask2.reshape(_B, _N, 1)
    thr16 = jnp.full((16,), mask_threshold, jnp.float32)
    thr1 = jnp.full((1,), mask_threshold, jnp.float32)
    x1 = _sc_route_x1(im1.reshape(_T, _C), im2.reshape(_T, _C), mask1, thr16)
    x2 = _tc_select_x2(im1, im2, m2col, thr1)
    return x1, x2
